# baseline (speedup 1.0000x reference)
import functools

import jax
import jax.numpy as jnp
from jax import lax
from jax.experimental import pallas as pl
from jax.experimental.pallas import tpu as pltpu
from jax.experimental.pallas import tpu_sc as plsc

N = 50000
E = 1600000
C = 32
EDGE_DIM = 4
H = 10

NC = 2
NS = 16
NW = NC * NS

BK = 512
SB = BK // 128
E_PAD = 1605632
EPT = E_PAD // NW
NIT = EPT // BK
E_ROWS = E_PAD // 128
RPT = EPT // 128

N_PAD = 51200
RN = N_PAD // NS

MLP_BLK = 16384
NBLK = E_PAD // MLP_BLK


def _mlp_body(ea_ref, w1_ref, b1_ref, w2_ref, b2_ref, d_ref, a_ref, b_ref):
    i = pl.program_id(0)
    ea = ea_ref[...].astype(jnp.bfloat16)
    w1 = w1_ref[...].astype(jnp.bfloat16)
    h = lax.dot_general(ea, w1, (((1,), (0,)), ((), ())),
                        preferred_element_type=jnp.float32)
    h = jnp.maximum(h + b1_ref[...][None], 0.0).astype(jnp.bfloat16)
    w2 = w2_ref[...].astype(jnp.bfloat16)
    c = lax.dot_general(h, w2, (((1,), (0,)), ((), ())),
                        preferred_element_type=jnp.float32)
    c = jnp.tanh(c + b2_ref[...][None])
    d2 = d_ref[0, 0] * d_ref[0, 0]
    eid = i * MLP_BLK + lax.broadcasted_iota(jnp.int32, (MLP_BLK, 1), 0)
    valid = eid < E
    av = jnp.where(valid, c[:, 0:1] + d2, 0.0)
    bv = jnp.where(valid, c[:, 1:2] - d2, 0.0)
    a_ref[...] = av.reshape(8, 2048)
    b_ref[...] = bv.reshape(8, 2048)


def _edge_coeffs(edge_attr, W1, b1, W2, b2, diff_param):
    return pl.pallas_call(
        _mlp_body,
        grid=(NBLK,),
        in_specs=[
            pl.BlockSpec((MLP_BLK, EDGE_DIM), lambda i: (i, 0)),
            pl.BlockSpec((EDGE_DIM, H), lambda i: (0, 0)),
            pl.BlockSpec((H,), lambda i: (0,)),
            pl.BlockSpec((H, 2), lambda i: (0, 0)),
            pl.BlockSpec((2,), lambda i: (0,)),
            pl.BlockSpec(memory_space=pltpu.SMEM),
        ],
        out_specs=[
            pl.BlockSpec((8, 2048), lambda i: (i, 0)),
            pl.BlockSpec((8, 2048), lambda i: (i, 0)),
        ],
        out_shape=[
            jax.ShapeDtypeStruct((NBLK * 8, 2048), jnp.float32),
            jax.ShapeDtypeStruct((NBLK * 8, 2048), jnp.float32),
        ],
    )(edge_attr, W1, b1, W2, b2, diff_param.reshape(1, 1))


def _sc_body(x_hbm, src_hbm, dst_hbm, a_hbm, b_hbm, zrows_hbm, z1_hbm,
             accp_hbm, sbp_hbm,
             acc_sh, sb_sh, idx_s, idx_d, b_b, rows, a_vm, sem, sem_i):
    cc = lax.axis_index("c")
    ss = lax.axis_index("s")
    wid = cc * NS + ss

    pltpu.sync_copy(zrows_hbm, acc_sh.at[pl.ds(ss * RN, RN)])
    pltpu.sync_copy(z1_hbm.at[pl.ds(ss * RN, RN)], sb_sh.at[pl.ds(ss * RN, RN)])
    plsc.subcore_barrier()

    base_row = wid * RPT
    HB = BK // 2
    HG = HB // 16

    def fire_inputs(i, p):
        r0 = base_row + i * SB
        pltpu.async_copy(src_hbm.at[pl.ds(r0, SB)], idx_s.at[p], sem_i)
        pltpu.async_copy(dst_hbm.at[pl.ds(r0, SB)], idx_d.at[p], sem_i)
        pltpu.async_copy(b_hbm.at[pl.ds(r0, SB)], b_b.at[p], sem_i)
        pltpu.async_copy(a_hbm.at[pl.ds(r0, SB)], a_vm.at[p], sem_i)

    def wait_inputs(p):
        pltpu.make_async_copy(src_hbm.at[pl.ds(0, SB)], idx_s.at[p], sem_i).wait()
        pltpu.make_async_copy(dst_hbm.at[pl.ds(0, SB)], idx_d.at[p], sem_i).wait()
        pltpu.make_async_copy(b_hbm.at[pl.ds(0, SB)], b_b.at[p], sem_i).wait()
        pltpu.make_async_copy(a_hbm.at[pl.ds(0, SB)], a_vm.at[p], sem_i).wait()

    def fire_gathers(p, h):
        for j in range(2):
            sj = h * 2 + j
            pltpu.async_copy(x_hbm.at[idx_d.at[p, sj]],
                             rows.at[pl.ds(sj * 128, 128)], sem)

    def wait_gathers(h):
        for j in range(2):
            sj = h * 2 + j
            pltpu.make_async_copy(x_hbm.at[pl.ds(0, 128)],
                                  rows.at[pl.ds(sj * 128, 128)], sem).wait()

    def scale_half(p, h):
        def grp(g, _):
            jj = g >> 3
            ll = (g & 7) * 16
            av = a_vm[p, jj, pl.ds(ll, 16)]
            base = g * 16
            for t in range(16):
                sc = av[t]
                k = base + t
                rows[k, pl.ds(0, 16)] = rows[k, pl.ds(0, 16)] * sc
                rows[k, pl.ds(16, 16)] = rows[k, pl.ds(16, 16)] * sc
            return 0
        lax.fori_loop(h * HG, (h + 1) * HG, grp, 0)

    def scatter_half(p, h):
        for j in range(2):
            sj = h * 2 + j
            pltpu.sync_copy(b_b.at[p, sj], sb_sh.at[idx_s.at[p, sj]], add=True)
            pltpu.sync_copy(rows.at[pl.ds(sj * 128, 128)],
                            acc_sh.at[idx_s.at[p, sj]], add=True)

    fire_inputs(0, 0)
    wait_inputs(0)
    fire_gathers(0, 0)

    def blk(i, carry):
        p = lax.rem(i, 2)
        q = lax.rem(i + 1, 2)

        @pl.when(i + 1 < NIT)
        def _():
            fire_inputs(i + 1, q)

        wait_gathers(0)
        fire_gathers(p, 1)
        scale_half(p, 0)
        scatter_half(p, 0)

        wait_gathers(1)

        @pl.when(i + 1 < NIT)
        def _():
            wait_inputs(q)
            fire_gathers(q, 0)

        scale_half(p, 1)
        scatter_half(p, 1)
        return 0

    lax.fori_loop(0, NIT, blk, 0)
    plsc.subcore_barrier()

    pltpu.sync_copy(acc_sh.at[pl.ds(ss * RN, RN)],
                    accp_hbm.at[cc, pl.ds(ss * RN, RN)])
    pltpu.sync_copy(sb_sh.at[pl.ds(ss * RN, RN)],
                    sbp_hbm.at[cc, pl.ds(ss * RN, RN)])


_sc_scatter = functools.partial(
    pl.kernel,
    out_type=[
        jax.ShapeDtypeStruct((NC, N_PAD, C), jnp.float32),
        jax.ShapeDtypeStruct((NC, N_PAD), jnp.float32),
    ],
    mesh=plsc.VectorSubcoreMesh(core_axis_name="c", subcore_axis_name="s"),
    compiler_params=pltpu.CompilerParams(needs_layout_passes=False,
                                         use_tc_tiling_on_sc=False),
    scratch_types=[
        pltpu.VMEM_SHARED((N_PAD, C), jnp.float32),
        pltpu.VMEM_SHARED((N_PAD,), jnp.float32),
        pltpu.VMEM((2, SB, 128), jnp.int32),
        pltpu.VMEM((2, SB, 128), jnp.int32),
        pltpu.VMEM((2, SB, 128), jnp.float32),
        pltpu.VMEM((BK, C), jnp.float32),
        pltpu.VMEM((2, SB, 128), jnp.float32),
        pltpu.SemaphoreType.DMA,
        pltpu.SemaphoreType.DMA,
    ],
)(_sc_body)


def _combine_body(x_ref, sb_ref, acc_ref, out_ref):
    acc = acc_ref[0] + acc_ref[1]
    r = lax.broadcasted_iota(jnp.int32, (C, C), 0)
    cidx = lax.broadcasted_iota(jnp.int32, (C, C), 1)
    eye = jnp.where(r == cidx, 1.0, 0.0).astype(jnp.float32)
    acc_t = lax.dot_general(eye, acc, (((1,), (1,)), ((), ())),
                            precision=lax.Precision.HIGHEST)
    sb = jnp.sum(sb_ref[...], axis=0, keepdims=True)
    out_ref[...] = x_ref[...] * (1.0 + sb) + acc_t


def _combine(x2, sbp, accp):
    BN = 1024
    return pl.pallas_call(
        _combine_body,
        grid=(pl.cdiv(N, BN),),
        in_specs=[
            pl.BlockSpec((C, BN), lambda i: (0, i)),
            pl.BlockSpec((NC, BN), lambda i: (0, i)),
            pl.BlockSpec((2, BN, C), lambda i: (0, i, 0)),
        ],
        out_specs=pl.BlockSpec((C, BN), lambda i: (0, i)),
        out_shape=jax.ShapeDtypeStruct((C, N), jnp.float32),
    )(x2, sbp, accp)


def kernel(x, edge_index, edge_attr, W1, b1, W2, b2, diff_param):
    x2 = x.reshape(C, N)
    x_nc = jnp.pad(x2.T, ((0, N_PAD - N), (0, 0)))

    npad = E_PAD - E
    src_p = jnp.concatenate(
        [edge_index[0], (jnp.arange(npad, dtype=jnp.int32) & 1023) + N])
    dst_p = jnp.pad(edge_index[1], (0, npad))
    src_r = src_p.reshape(E_ROWS, 128)
    dst_r = dst_p.reshape(E_ROWS, 128)

    a_r, b_r = _edge_coeffs(edge_attr, W1, b1, W2, b2, diff_param)
    a2 = a_r.reshape(E_ROWS, 128)
    b2_ = b_r.reshape(E_ROWS, 128)

    zrows = jnp.zeros((RN, C), jnp.float32)
    z1 = jnp.zeros((N_PAD,), jnp.float32)

    accp, sbp = _sc_scatter(x_nc, src_r, dst_r, a2, b2_, zrows, z1)

    out2 = _combine(x2, sbp, accp)
    return out2.reshape(1, C, N)


# speedup vs baseline: 2.3583x; 2.3583x over previous
import functools

import jax
import jax.numpy as jnp
from jax import lax
from jax.experimental import pallas as pl
from jax.experimental.pallas import tpu as pltpu
from jax.experimental.pallas import tpu_sc as plsc

N = 50000
E = 1600000
C = 32
EDGE_DIM = 4
H = 10

NC = 2
NS = 16
NW = NC * NS

BK = 512
SB = BK // 128
E_PAD = 1605632
EPT = E_PAD // NW
NIT = EPT // BK
E_ROWS = E_PAD // 128
RPT = EPT // 128

N_PAD = 51200
RN = N_PAD // NS

MLP_BLK = 16384
NBLK = E_PAD // MLP_BLK


def _round_bf16(v):
    return v.astype(jnp.bfloat16).astype(jnp.float32)


def _mlp_body(ea_ref, w1_ref, b1_ref, w2_ref, b2_ref, d_ref, a_ref, b_ref):
    ea = _round_bf16(ea_ref[...][:, 0])
    hs = []
    for j in range(H):
        h = b1_ref[j]
        for k in range(EDGE_DIM):
            h = h + ea[k] * _round_bf16(w1_ref[k, j])
        hs.append(_round_bf16(jnp.maximum(h, 0.0)))
    c0 = b2_ref[0]
    c1 = b2_ref[1]
    for j in range(H):
        c0 = c0 + hs[j] * _round_bf16(w2_ref[j, 0])
        c1 = c1 + hs[j] * _round_bf16(w2_ref[j, 1])
    d2 = d_ref[0] * d_ref[0]
    a_ref[...] = (jnp.tanh(c0) + d2)[None]
    b_ref[...] = (jnp.tanh(c1) - d2)[None]


def _edge_coeffs(ea_r, W1, b1, W2, b2, diff_param):
    smem = pl.BlockSpec(memory_space=pltpu.SMEM)
    return pl.pallas_call(
        _mlp_body,
        grid=(NBLK,),
        in_specs=[
            pl.BlockSpec((EDGE_DIM, 1, 8, 2048), lambda i: (0, i, 0, 0)),
            smem, smem, smem, smem, smem,
        ],
        out_specs=[
            pl.BlockSpec((1, 8, 2048), lambda i: (i, 0, 0)),
            pl.BlockSpec((1, 8, 2048), lambda i: (i, 0, 0)),
        ],
        out_shape=[
            jax.ShapeDtypeStruct((NBLK, 8, 2048), jnp.float32),
            jax.ShapeDtypeStruct((NBLK, 8, 2048), jnp.float32),
        ],
    )(ea_r, W1, b1, W2, b2, diff_param)


def _sc_body(x_hbm, src_hbm, dst_hbm, a_hbm, b_hbm, zrows_hbm, z1_hbm,
             accp_hbm, sbp_hbm,
             acc_sh, sb_sh, idx_s, idx_d, b_b, rows, a_vm, sem, sem_i):
    cc = lax.axis_index("c")
    ss = lax.axis_index("s")
    wid = cc * NS + ss

    pltpu.sync_copy(zrows_hbm, acc_sh.at[pl.ds(ss * RN, RN)])
    pltpu.sync_copy(z1_hbm.at[pl.ds(ss * RN, RN)], sb_sh.at[pl.ds(ss * RN, RN)])
    plsc.subcore_barrier()

    R0 = 436
    R1 = 2 * RPT - R0
    base_row = lax.select(cc == 0, ss * R0, NS * R0 + ss * R1)
    nit = lax.select(cc == 0, R0 // SB, R1 // SB)
    HB = BK // 2
    HG = HB // 16

    def fire_inputs(i, p):
        r0 = base_row + i * SB
        pltpu.async_copy(src_hbm.at[pl.ds(r0, SB)], idx_s.at[p], sem_i)
        pltpu.async_copy(dst_hbm.at[pl.ds(r0, SB)], idx_d.at[p], sem_i)
        pltpu.async_copy(b_hbm.at[pl.ds(r0, SB)], b_b.at[p], sem_i)
        pltpu.async_copy(a_hbm.at[pl.ds(r0, SB)], a_vm.at[p], sem_i)

    def wait_inputs(p):
        pltpu.make_async_copy(src_hbm.at[pl.ds(0, SB)], idx_s.at[p], sem_i).wait()
        pltpu.make_async_copy(dst_hbm.at[pl.ds(0, SB)], idx_d.at[p], sem_i).wait()
        pltpu.make_async_copy(b_hbm.at[pl.ds(0, SB)], b_b.at[p], sem_i).wait()
        pltpu.make_async_copy(a_hbm.at[pl.ds(0, SB)], a_vm.at[p], sem_i).wait()

    def fire_gathers(p, h):
        for j in range(2):
            sj = h * 2 + j
            pltpu.async_copy(x_hbm.at[idx_d.at[p, sj]],
                             rows.at[pl.ds(sj * 128, 128)], sem)

    def wait_gathers(h):
        for j in range(2):
            sj = h * 2 + j
            pltpu.make_async_copy(x_hbm.at[pl.ds(0, 128)],
                                  rows.at[pl.ds(sj * 128, 128)], sem).wait()

    def scale_half(p, h):
        def grp(g, _):
            jj = g >> 3
            ll = (g & 7) * 16
            av = a_vm[p, jj, pl.ds(ll, 16)]
            base = g * 16
            for t in range(16):
                sc = av[t]
                k = base + t
                rows[k, pl.ds(0, 16)] = rows[k, pl.ds(0, 16)] * sc
                rows[k, pl.ds(16, 16)] = rows[k, pl.ds(16, 16)] * sc
            return 0
        lax.fori_loop(h * HG, (h + 1) * HG, grp, 0)

    def scatter_half(p, h):
        for j in range(2):
            sj = h * 2 + j
            pltpu.sync_copy(b_b.at[p, sj], sb_sh.at[idx_s.at[p, sj]], add=True)
            pltpu.sync_copy(rows.at[pl.ds(sj * 128, 128)],
                            acc_sh.at[idx_s.at[p, sj]], add=True)

    fire_inputs(0, 0)
    wait_inputs(0)
    fire_gathers(0, 0)

    def blk(i, carry):
        p = lax.rem(i, 2)
        q = lax.rem(i + 1, 2)

        @pl.when(i + 1 < nit)
        def _():
            fire_inputs(i + 1, q)

        wait_gathers(0)
        fire_gathers(p, 1)
        scale_half(p, 0)
        scatter_half(p, 0)

        wait_gathers(1)

        @pl.when(i + 1 < nit)
        def _():
            wait_inputs(q)
            fire_gathers(q, 0)

        scale_half(p, 1)
        scatter_half(p, 1)
        return 0

    lax.fori_loop(0, nit, blk, 0)
    plsc.subcore_barrier()

    pltpu.sync_copy(acc_sh.at[pl.ds(ss * RN, RN)],
                    accp_hbm.at[cc, pl.ds(ss * RN, RN)])
    pltpu.sync_copy(sb_sh.at[pl.ds(ss * RN, RN)],
                    sbp_hbm.at[cc, pl.ds(ss * RN, RN)])


_sc_scatter = functools.partial(
    pl.kernel,
    out_type=[
        jax.ShapeDtypeStruct((NC, N_PAD, C), jnp.float32),
        jax.ShapeDtypeStruct((NC, N_PAD), jnp.float32),
    ],
    mesh=plsc.VectorSubcoreMesh(core_axis_name="c", subcore_axis_name="s"),
    compiler_params=pltpu.CompilerParams(needs_layout_passes=False,
                                         use_tc_tiling_on_sc=False),
    scratch_types=[
        pltpu.VMEM_SHARED((N_PAD, C), jnp.float32),
        pltpu.VMEM_SHARED((N_PAD,), jnp.float32),
        pltpu.VMEM((2, SB, 128), jnp.int32),
        pltpu.VMEM((2, SB, 128), jnp.int32),
        pltpu.VMEM((2, SB, 128), jnp.float32),
        pltpu.VMEM((BK, C), jnp.float32),
        pltpu.VMEM((2, SB, 128), jnp.float32),
        pltpu.SemaphoreType.DMA,
        pltpu.SemaphoreType.DMA,
    ],
)(_sc_body)


def _combine_body(x_ref, sb_ref, acc_ref, out_ref):
    acc = acc_ref[0] + acc_ref[1]
    r = lax.broadcasted_iota(jnp.int32, (C, C), 0)
    cidx = lax.broadcasted_iota(jnp.int32, (C, C), 1)
    eye = jnp.where(r == cidx, 1.0, 0.0).astype(jnp.float32)
    acc_t = lax.dot_general(eye, acc, (((1,), (1,)), ((), ())),
                            precision=lax.Precision.HIGHEST)
    sb = jnp.sum(sb_ref[...], axis=0, keepdims=True)
    out_ref[...] = x_ref[...] * (1.0 + sb) + acc_t


def _combine(x2, sbp, accp):
    BN = 1024
    return pl.pallas_call(
        _combine_body,
        grid=(pl.cdiv(N, BN),),
        in_specs=[
            pl.BlockSpec((C, BN), lambda i: (0, i)),
            pl.BlockSpec((NC, BN), lambda i: (0, i)),
            pl.BlockSpec((2, BN, C), lambda i: (0, i, 0)),
        ],
        out_specs=pl.BlockSpec((C, BN), lambda i: (0, i)),
        out_shape=jax.ShapeDtypeStruct((C, N), jnp.float32),
    )(x2, sbp, accp)


def kernel(x, edge_index, edge_attr, W1, b1, W2, b2, diff_param):
    x2 = x.reshape(C, N)
    x_nc = jnp.pad(x2.T, ((0, N_PAD - N), (0, 0)))

    npad = E_PAD - E
    src_p = jnp.concatenate(
        [edge_index[0], (jnp.arange(npad, dtype=jnp.int32) & 1023) + N])
    dst_p = jnp.pad(edge_index[1], (0, npad))
    src_r = src_p.reshape(E_ROWS, 128)
    dst_r = dst_p.reshape(E_ROWS, 128)

    ea_r = jnp.pad(edge_attr, ((0, npad), (0, 0))).T.reshape(
        EDGE_DIM, NBLK, 8, 2048)
    a_r, b_r = _edge_coeffs(ea_r, W1, b1, W2, b2, diff_param)
    a2 = a_r.reshape(E_ROWS, 128)
    b2_ = b_r.reshape(E_ROWS, 128)

    zrows = jnp.zeros((RN, C), jnp.float32)
    z1 = jnp.zeros((N_PAD,), jnp.float32)

    accp, sbp = _sc_scatter(x_nc, src_r, dst_r, a2, b2_, zrows, z1)

    out2 = _combine(x2, sbp, accp)
    return out2.reshape(1, C, N)


# speedup vs baseline: 2.4110x; 1.0224x over previous
import functools

import jax
import jax.numpy as jnp
from jax import lax
from jax.experimental import pallas as pl
from jax.experimental.pallas import tpu as pltpu
from jax.experimental.pallas import tpu_sc as plsc

N = 50000
E = 1600000
C = 32
EDGE_DIM = 4
H = 10

NC = 2
NS = 16
NW = NC * NS

BK = 512
SB = BK // 128
E_PAD = 1605632
EPT = E_PAD // NW
NIT = EPT // BK
E_ROWS = E_PAD // 128
RPT = EPT // 128

N_PAD = 51200
RN = N_PAD // NS

MLP_BLK = 16384
NBLK = E_PAD // MLP_BLK


def _round_bf16(v):
    return v.astype(jnp.bfloat16).astype(jnp.float32)


def _mlp_body(ea_ref, w1_ref, b1_ref, w2_ref, b2_ref, d_ref, a_ref, b_ref):
    ea = _round_bf16(ea_ref[...][:, 0])
    hs = []
    for j in range(H):
        h = b1_ref[j]
        for k in range(EDGE_DIM):
            h = h + ea[k] * _round_bf16(w1_ref[k, j])
        hs.append(_round_bf16(jnp.maximum(h, 0.0)))
    c0 = b2_ref[0]
    c1 = b2_ref[1]
    for j in range(H):
        c0 = c0 + hs[j] * _round_bf16(w2_ref[j, 0])
        c1 = c1 + hs[j] * _round_bf16(w2_ref[j, 1])
    d2 = d_ref[0] * d_ref[0]
    a_ref[...] = (jnp.tanh(c0) + d2)[None]
    b_ref[...] = (jnp.tanh(c1) - d2)[None]


def _edge_coeffs(ea_r, W1, b1, W2, b2, diff_param):
    smem = pl.BlockSpec(memory_space=pltpu.SMEM)
    return pl.pallas_call(
        _mlp_body,
        grid=(NBLK,),
        in_specs=[
            pl.BlockSpec((EDGE_DIM, 1, 8, 2048), lambda i: (0, i, 0, 0)),
            smem, smem, smem, smem, smem,
        ],
        out_specs=[
            pl.BlockSpec((1, 8, 2048), lambda i: (i, 0, 0)),
            pl.BlockSpec((1, 8, 2048), lambda i: (i, 0, 0)),
        ],
        out_shape=[
            jax.ShapeDtypeStruct((NBLK, 8, 2048), jnp.float32),
            jax.ShapeDtypeStruct((NBLK, 8, 2048), jnp.float32),
        ],
    )(ea_r, W1, b1, W2, b2, diff_param)


def _sc_body(x_hbm, src_hbm, dst_hbm, a_hbm, b_hbm, zrows_hbm, z1_hbm,
             accp_hbm, sbp_hbm,
             acc_sh, sb_sh, idx_s, idx_d, b_b, rows, a_vm, sem, sem_i):
    cc = lax.axis_index("c")
    ss = lax.axis_index("s")
    wid = cc * NS + ss

    pltpu.sync_copy(zrows_hbm, acc_sh.at[pl.ds(ss * RN, RN)])
    pltpu.sync_copy(z1_hbm.at[pl.ds(ss * RN, RN)], sb_sh.at[pl.ds(ss * RN, RN)])
    plsc.subcore_barrier()

    R0 = 444
    R1 = 2 * RPT - R0
    base_row = lax.select(cc == 0, ss * R0, NS * R0 + ss * R1)
    nit = lax.select(cc == 0, R0 // SB, R1 // SB)
    HB = BK // 2
    HG = HB // 16

    def fire_inputs(i, p):
        r0 = base_row + i * SB
        pltpu.async_copy(src_hbm.at[pl.ds(r0, SB)], idx_s.at[p], sem_i)
        pltpu.async_copy(dst_hbm.at[pl.ds(r0, SB)], idx_d.at[p], sem_i)
        pltpu.async_copy(b_hbm.at[pl.ds(r0, SB)], b_b.at[p], sem_i)
        pltpu.async_copy(a_hbm.at[pl.ds(r0, SB)], a_vm.at[p], sem_i)

    def wait_inputs(p):
        pltpu.make_async_copy(src_hbm.at[pl.ds(0, SB)], idx_s.at[p], sem_i).wait()
        pltpu.make_async_copy(dst_hbm.at[pl.ds(0, SB)], idx_d.at[p], sem_i).wait()
        pltpu.make_async_copy(b_hbm.at[pl.ds(0, SB)], b_b.at[p], sem_i).wait()
        pltpu.make_async_copy(a_hbm.at[pl.ds(0, SB)], a_vm.at[p], sem_i).wait()

    def fire_gathers(p, h):
        for j in range(2):
            sj = h * 2 + j
            pltpu.async_copy(x_hbm.at[idx_d.at[p, sj]],
                             rows.at[pl.ds(sj * 128, 128)], sem)

    def wait_gathers(h):
        for j in range(2):
            sj = h * 2 + j
            pltpu.make_async_copy(x_hbm.at[pl.ds(0, 128)],
                                  rows.at[pl.ds(sj * 128, 128)], sem).wait()

    def scale_half(p, h):
        @plsc.parallel_loop(h * HG, (h + 1) * HG, unroll=2)
        def _(g):
            jj = g >> 3
            ll = (g & 7) * 16
            av = a_vm[p, jj, pl.ds(ll, 16)]
            base = g * 16
            for t in range(16):
                sc = av[t]
                k = base + t
                rows[k, pl.ds(0, 16)] = rows[k, pl.ds(0, 16)] * sc
                rows[k, pl.ds(16, 16)] = rows[k, pl.ds(16, 16)] * sc

    def scatter_half(p, h):
        for j in range(2):
            sj = h * 2 + j
            pltpu.sync_copy(b_b.at[p, sj], sb_sh.at[idx_s.at[p, sj]], add=True)
            pltpu.sync_copy(rows.at[pl.ds(sj * 128, 128)],
                            acc_sh.at[idx_s.at[p, sj]], add=True)

    fire_inputs(0, 0)
    wait_inputs(0)
    fire_gathers(0, 0)

    def blk(i, carry):
        p = lax.rem(i, 2)
        q = lax.rem(i + 1, 2)

        @pl.when(i + 1 < nit)
        def _():
            fire_inputs(i + 1, q)

        wait_gathers(0)
        fire_gathers(p, 1)
        scale_half(p, 0)
        scatter_half(p, 0)

        wait_gathers(1)

        @pl.when(i + 1 < nit)
        def _():
            wait_inputs(q)
            fire_gathers(q, 0)

        scale_half(p, 1)
        scatter_half(p, 1)
        return 0

    lax.fori_loop(0, nit, blk, 0)
    plsc.subcore_barrier()

    pltpu.sync_copy(acc_sh.at[pl.ds(ss * RN, RN)],
                    accp_hbm.at[cc, pl.ds(ss * RN, RN)])
    pltpu.sync_copy(sb_sh.at[pl.ds(ss * RN, RN)],
                    sbp_hbm.at[cc, pl.ds(ss * RN, RN)])


_sc_scatter = functools.partial(
    pl.kernel,
    out_type=[
        jax.ShapeDtypeStruct((NC, N_PAD, C), jnp.float32),
        jax.ShapeDtypeStruct((NC, N_PAD), jnp.float32),
    ],
    mesh=plsc.VectorSubcoreMesh(core_axis_name="c", subcore_axis_name="s"),
    compiler_params=pltpu.CompilerParams(needs_layout_passes=False,
                                         use_tc_tiling_on_sc=False),
    scratch_types=[
        pltpu.VMEM_SHARED((N_PAD, C), jnp.float32),
        pltpu.VMEM_SHARED((N_PAD,), jnp.float32),
        pltpu.VMEM((2, SB, 128), jnp.int32),
        pltpu.VMEM((2, SB, 128), jnp.int32),
        pltpu.VMEM((2, SB, 128), jnp.float32),
        pltpu.VMEM((BK, C), jnp.float32),
        pltpu.VMEM((2, SB, 128), jnp.float32),
        pltpu.SemaphoreType.DMA,
        pltpu.SemaphoreType.DMA,
    ],
)(_sc_body)


def _combine_body(x_ref, sb_ref, acc_ref, out_ref):
    acc = acc_ref[0] + acc_ref[1]
    r = lax.broadcasted_iota(jnp.int32, (C, C), 0)
    cidx = lax.broadcasted_iota(jnp.int32, (C, C), 1)
    eye = jnp.where(r == cidx, 1.0, 0.0).astype(jnp.float32)
    acc_t = lax.dot_general(eye, acc, (((1,), (1,)), ((), ())),
                            precision=lax.Precision.HIGHEST)
    sb = jnp.sum(sb_ref[...], axis=0, keepdims=True)
    out_ref[...] = x_ref[...] * (1.0 + sb) + acc_t


def _combine(x2, sbp, accp):
    BN = 1024
    return pl.pallas_call(
        _combine_body,
        grid=(pl.cdiv(N, BN),),
        in_specs=[
            pl.BlockSpec((C, BN), lambda i: (0, i)),
            pl.BlockSpec((NC, BN), lambda i: (0, i)),
            pl.BlockSpec((2, BN, C), lambda i: (0, i, 0)),
        ],
        out_specs=pl.BlockSpec((C, BN), lambda i: (0, i)),
        out_shape=jax.ShapeDtypeStruct((C, N), jnp.float32),
    )(x2, sbp, accp)


def kernel(x, edge_index, edge_attr, W1, b1, W2, b2, diff_param):
    x2 = x.reshape(C, N)
    x_nc = jnp.pad(x2.T, ((0, N_PAD - N), (0, 0)))

    npad = E_PAD - E
    src_p = jnp.concatenate(
        [edge_index[0], (jnp.arange(npad, dtype=jnp.int32) & 1023) + N])
    dst_p = jnp.pad(edge_index[1], (0, npad))
    src_r = src_p.reshape(E_ROWS, 128)
    dst_r = dst_p.reshape(E_ROWS, 128)

    ea_r = jnp.pad(edge_attr, ((0, npad), (0, 0))).T.reshape(
        EDGE_DIM, NBLK, 8, 2048)
    a_r, b_r = _edge_coeffs(ea_r, W1, b1, W2, b2, diff_param)
    a2 = a_r.reshape(E_ROWS, 128)
    b2_ = b_r.reshape(E_ROWS, 128)

    zrows = jnp.zeros((RN, C), jnp.float32)
    z1 = jnp.zeros((N_PAD,), jnp.float32)

    accp, sbp = _sc_scatter(x_nc, src_r, dst_r, a2, b2_, zrows, z1)

    out2 = _combine(x2, sbp, accp)
    return out2.reshape(1, C, N)
